# baseline (speedup 1.0000x reference)
import functools

import jax
import jax.numpy as jnp
from jax import lax
from jax.experimental import pallas as pl
from jax.experimental.pallas import tpu as pltpu
from jax.experimental.pallas import tpu_sc as plsc

D = 32
NF = 26
NFP = 32
BATCH = 16384
NC, NS = 2, 16
NW = NC * NS
BW = BATCH // NW
NJ = BW // 16

_mesh = plsc.VectorSubcoreMesh(core_axis_name="c", subcore_axis_name="s")


@functools.partial(
    pl.kernel,
    mesh=_mesh,
    out_type=jax.ShapeDtypeStruct((NF, D, BATCH), jnp.float32),
    scratch_types=[
        pltpu.VMEM((NFP, BW), jnp.int32),
        pltpu.VMEM((BW, D), jnp.float32),
        pltpu.VMEM((BW, D), jnp.float32),
        pltpu.VMEM((D, BW), jnp.float32),
        pltpu.VMEM((D, BW), jnp.float32),
        pltpu.SemaphoreType.DMA,
        pltpu.SemaphoreType.DMA,
        pltpu.SemaphoreType.DMA,
        pltpu.SemaphoreType.DMA,
    ],
    compiler_params=pltpu.CompilerParams(use_tc_tiling_on_sc=False,
                                         needs_layout_passes=False),
)
def _gather_rows(xT_hbm, w_hbm, out_hbm, idx_v, rows0, rows1, tr0, tr1,
                 s_g0, s_g1, s_o0, s_o1):
    wid = lax.axis_index("s") * NC + lax.axis_index("c")
    b0 = wid * BW

    rows = (rows0, rows1)
    trs = (tr0, tr1)
    s_g = (s_g0, s_g1)
    s_o = (s_o0, s_o1)

    pltpu.sync_copy(xT_hbm.at[:, pl.ds(b0, BW)], idx_v)

    def gather(f, b):
        return pltpu.async_copy(w_hbm.at[idx_v.at[f]], rows[b], s_g[b])

    def writeback(f, b):
        return pltpu.async_copy(trs[b], out_hbm.at[f, :, pl.ds(b0, BW)],
                                s_o[b])

    iota16 = lax.iota(jnp.int32, 16)

    def transpose(b):
        src, dst = rows[b], trs[b]

        def body_j(j, carry):
            row_idx = iota16 + j * 16
            for d in range(D):
                col_idx = jnp.full((16,), d, jnp.int32)
                v = plsc.load_gather(src, [row_idx, col_idx])
                dst[d, pl.ds(j * 16, 16)] = v
            return carry

        lax.fori_loop(0, NJ, body_j, 0)

    h_g = [None] * NF
    h_o = [None] * NF
    h_g[0] = gather(0, 0)
    for f in range(NF):
        b = f % 2
        h_g[f].wait()
        if f + 1 < NF:
            h_g[f + 1] = gather(f + 1, 1 - b)
        if f >= 2:
            h_o[f - 2].wait()
        transpose(b)
        h_o[f] = writeback(f, b)
    h_o[NF - 2].wait()
    h_o[NF - 1].wait()


def kernel(x, weight):
    xp = jnp.pad(x, ((0, 0), (0, NFP - NF))).T
    out = _gather_rows(xp, weight)
    return out.transpose(2, 0, 1)


# speedup vs baseline: 1.3785x; 1.3785x over previous
import functools

import jax
import jax.numpy as jnp
from jax import lax
from jax.experimental import pallas as pl
from jax.experimental.pallas import tpu as pltpu
from jax.experimental.pallas import tpu_sc as plsc

D = 32
NF = 26
BATCH = 16384
NC, NS = 2, 16
NW = NC * NS
BW = BATCH // NW
CH = 128
NCHUNK = NF * (BW // CH)
PITCH = 129

_mesh = plsc.VectorSubcoreMesh(core_axis_name="c", subcore_axis_name="s")


@functools.partial(
    pl.kernel,
    mesh=_mesh,
    out_type=jax.ShapeDtypeStruct((NF, D, BATCH), jnp.float32),
    scratch_types=[
        pltpu.VMEM((4, 4, 8, CH), jnp.int32),
        pltpu.VMEM((CH, D), jnp.float32),
        pltpu.VMEM((CH, D), jnp.float32),
        pltpu.VMEM((D, PITCH), jnp.float32),
        pltpu.VMEM((D, PITCH), jnp.float32),
        pltpu.SemaphoreType.DMA,
        pltpu.SemaphoreType.DMA,
        pltpu.SemaphoreType.DMA,
        pltpu.SemaphoreType.DMA,
    ],
    compiler_params=pltpu.CompilerParams(use_tc_tiling_on_sc=False,
                                         needs_layout_passes=False),
)
def _gather_rows(xq_hbm, w_hbm, out_hbm, idx_raw, rows0, rows1, tr0, tr1,
                 s_g0, s_g1, s_o0, s_o1):
    wid = lax.axis_index("s") * NC + lax.axis_index("c")
    b0 = wid * BW
    tcs = wid * (BW // CH)

    rows = (rows0, rows1)
    trs = (tr0, tr1)
    s_g = (s_g0, s_g1)
    s_o = (s_o0, s_o1)

    pltpu.sync_copy(xq_hbm.at[:, pl.ds(tcs, 4), :, :], idx_raw)

    iota16 = lax.iota(jnp.int32, 16)

    def split(i):
        f = i // 4
        tcl = lax.rem(i, 4)
        return f, tcl, f // 8, lax.rem(f, 8)

    def issue_gather(i, par):
        _, tcl, tr, r = split(i)
        return pltpu.async_copy(
            w_hbm.at[idx_raw.at[tr, tcl, r]], rows[par], s_g[par])

    def issue_write(i, par):
        f, tcl, _, _ = split(i)
        return pltpu.async_copy(
            trs[par].at[:, pl.ds(0, CH)],
            out_hbm.at[f, :, pl.ds(b0 + tcl * CH, CH)], s_o[par])

    def transpose(par):
        src, dst = rows[par], trs[par]

        def body_j(j, carry):
            for jj in range(8):
                row = j * 8 + jj
                rvec = jnp.full((16,), row, jnp.int32)
                v_lo = src[row, pl.ds(0, 16)]
                v_hi = src[row, pl.ds(16, 16)]
                plsc.store_scatter(dst, [iota16, rvec], v_lo)
                plsc.store_scatter(dst, [iota16 + 16, rvec], v_hi)
            return carry

        lax.fori_loop(0, CH // 8, body_j, 0)

    issue_gather(0, 0)
    issue_gather(1, 1)

    def step(k, carry):
        for par in range(2):
            i = 2 * k + par
            pltpu.make_async_copy(w_hbm.at[idx_raw.at[0, 0, 0]],
                                  rows[par], s_g[par]).wait()

            @pl.when(i >= 2)
            def _():
                pltpu.make_async_copy(
                    trs[par].at[:, pl.ds(0, CH)],
                    out_hbm.at[0, :, pl.ds(0, CH)], s_o[par]).wait()

            transpose(par)
            issue_write(i, par)

            @pl.when(i + 2 < NCHUNK)
            def _():
                issue_gather(i + 2, par)
        return carry

    lax.fori_loop(0, NCHUNK // 2, step, 0)

    for par in range(2):
        pltpu.make_async_copy(trs[par].at[:, pl.ds(0, CH)],
                              out_hbm.at[0, :, pl.ds(0, CH)], s_o[par]).wait()


def kernel(x, weight):
    xpT = jnp.pad(x, ((0, 0), (0, 32 - NF))).T
    xq = xpT.reshape(4, 8, CH, CH).transpose(0, 2, 1, 3)
    out = _gather_rows(xq, weight)
    return out.transpose(2, 0, 1)


# speedup vs baseline: 1.3815x; 1.0021x over previous
import functools

import jax
import jax.numpy as jnp
from jax import lax
from jax.experimental import pallas as pl
from jax.experimental.pallas import tpu as pltpu
from jax.experimental.pallas import tpu_sc as plsc

D = 32
NF = 26
BATCH = 16384
NC, NS = 2, 16
NW = NC * NS
BW = BATCH // NW
CH = 128
NCHUNK = NF * (BW // CH)
PITCH = 129

_mesh = plsc.VectorSubcoreMesh(core_axis_name="c", subcore_axis_name="s")


@functools.partial(
    pl.kernel,
    mesh=_mesh,
    out_type=jax.ShapeDtypeStruct((4, CH, 8, CH), jnp.int32),
    scratch_types=[],
    compiler_params=pltpu.CompilerParams(use_tc_tiling_on_sc=True),
)
def _stage_x(xT_hbm, xq_hbm):
    wid = lax.axis_index("s") * NC + lax.axis_index("c")
    for tr in range(4):
        rows = 8 if tr < 3 else NF - 24
        for tcl in range(4):
            tc = wid * 4 + tcl
            pltpu.sync_copy(
                xT_hbm.at[pl.ds(8 * tr, rows),
                          pl.ds(BW * wid + CH * tcl, CH)],
                xq_hbm.at[tr, tc, pl.ds(0, rows), :])


@functools.partial(
    pl.kernel,
    mesh=_mesh,
    out_type=jax.ShapeDtypeStruct((NF, D, BATCH), jnp.float32),
    scratch_types=[
        pltpu.VMEM((4, 4, 8, CH), jnp.int32),
        pltpu.VMEM((CH, D), jnp.float32),
        pltpu.VMEM((CH, D), jnp.float32),
        pltpu.VMEM((D, PITCH), jnp.float32),
        pltpu.VMEM((D, PITCH), jnp.float32),
        pltpu.SemaphoreType.DMA,
        pltpu.SemaphoreType.DMA,
        pltpu.SemaphoreType.DMA,
        pltpu.SemaphoreType.DMA,
    ],
    compiler_params=pltpu.CompilerParams(use_tc_tiling_on_sc=False,
                                         needs_layout_passes=False),
)
def _gather_rows(xq_hbm, w_hbm, out_hbm, idx_raw, rows0, rows1, tr0, tr1,
                 s_g0, s_g1, s_o0, s_o1):
    wid = lax.axis_index("s") * NC + lax.axis_index("c")
    b0 = wid * BW
    tcs = wid * (BW // CH)

    rows = (rows0, rows1)
    trs = (tr0, tr1)
    s_g = (s_g0, s_g1)
    s_o = (s_o0, s_o1)

    pltpu.sync_copy(xq_hbm.at[:, pl.ds(tcs, 4), :, :], idx_raw)

    iota16 = lax.iota(jnp.int32, 16)

    def split(i):
        f = i // 4
        tcl = lax.rem(i, 4)
        return f, tcl, f // 8, lax.rem(f, 8)

    def issue_gather(i, par):
        _, tcl, tr, r = split(i)
        return pltpu.async_copy(
            w_hbm.at[idx_raw.at[tr, tcl, r]], rows[par], s_g[par])

    def issue_write(i, par):
        f, tcl, _, _ = split(i)
        return pltpu.async_copy(
            trs[par].at[:, pl.ds(0, CH)],
            out_hbm.at[f, :, pl.ds(b0 + tcl * CH, CH)], s_o[par])

    def transpose(par):
        src, dst = rows[par], trs[par]

        def body_j(j, carry):
            for jj in range(8):
                row = j * 8 + jj
                rvec = jnp.full((16,), row, jnp.int32)
                v_lo = src[row, pl.ds(0, 16)]
                v_hi = src[row, pl.ds(16, 16)]
                plsc.store_scatter(dst, [iota16, rvec], v_lo)
                plsc.store_scatter(dst, [iota16 + 16, rvec], v_hi)
            return carry

        lax.fori_loop(0, CH // 8, body_j, 0)

    issue_gather(0, 0)
    issue_gather(1, 1)

    def step(k, carry):
        for par in range(2):
            i = 2 * k + par
            pltpu.make_async_copy(w_hbm.at[idx_raw.at[0, 0, 0]],
                                  rows[par], s_g[par]).wait()

            @pl.when(i >= 2)
            def _():
                pltpu.make_async_copy(
                    trs[par].at[:, pl.ds(0, CH)],
                    out_hbm.at[0, :, pl.ds(0, CH)], s_o[par]).wait()

            transpose(par)
            issue_write(i, par)

            @pl.when(i + 2 < NCHUNK)
            def _():
                issue_gather(i + 2, par)
        return carry

    lax.fori_loop(0, NCHUNK // 2, step, 0)

    for par in range(2):
        pltpu.make_async_copy(trs[par].at[:, pl.ds(0, CH)],
                              out_hbm.at[0, :, pl.ds(0, CH)], s_o[par]).wait()


def kernel(x, weight):
    xq = _stage_x(x.T)
    out = _gather_rows(xq, weight)
    return out.transpose(2, 0, 1)
